# baseline (speedup 1.0000x reference)
import functools

import jax
import jax.numpy as jnp
from jax.experimental import pallas as pl
from jax.experimental.pallas import tpu as pltpu

B, S, H, E, I, TOP_K = 8, 16, 1024, 16, 2048, 2
T = B * S
TI = 1024
NI = I // TI


def _moe_kernel(x_ref, rw_ref, wg_ref, wu_ref, w2_ref, out_ref, combine_ref):
    e = pl.program_id(0)
    ti = pl.program_id(1)

    @pl.when(jnp.logical_and(e == 0, ti == 0))
    def _routing():
        x = x_ref[...]
        logits = jax.lax.dot_general(
            x, rw_ref[...], (((1,), (1,)), ((), ())),
            preferred_element_type=jnp.float32)
        m = jnp.max(logits, axis=-1, keepdims=True)
        unnorm = jnp.exp(logits - m)
        probs = unnorm / jnp.sum(unnorm, axis=-1, keepdims=True)
        iota = jax.lax.broadcasted_iota(jnp.int32, (T, E), 1)
        m1 = jnp.max(probs, axis=-1, keepdims=True)
        i1 = jnp.min(jnp.where(probs == m1, iota, E), axis=-1, keepdims=True)
        sel1 = iota == i1
        probs2 = jnp.where(sel1, -jnp.inf, probs)
        m2 = jnp.max(probs2, axis=-1, keepdims=True)
        i2 = jnp.min(jnp.where(probs2 == m2, iota, E), axis=-1, keepdims=True)
        sel = sel1 | (iota == i2)
        combine_ref[...] = jnp.where(sel, probs, 0.0)
        out_ref[...] = jnp.zeros_like(out_ref)

    x = x_ref[...].astype(jnp.bfloat16)
    g = jax.lax.dot_general(x, wg_ref[0, 0].astype(jnp.bfloat16),
                            (((1,), (1,)), ((), ())),
                            preferred_element_type=jnp.float32)
    u = jax.lax.dot_general(x, wu_ref[0, 0].astype(jnp.bfloat16),
                            (((1,), (1,)), ((), ())),
                            preferred_element_type=jnp.float32)
    a = (g * jax.nn.sigmoid(g)) * u
    iota = jax.lax.broadcasted_iota(jnp.int32, (T, E), 1)
    col = jnp.sum(jnp.where(iota == e, combine_ref[...], 0.0), axis=1,
                  keepdims=True)
    a = (a * col).astype(jnp.bfloat16)
    o = jax.lax.dot_general(a, w2_ref[0].astype(jnp.bfloat16),
                            (((1,), (1,)), ((), ())),
                            preferred_element_type=jnp.float32)
    out_ref[...] += o


@jax.jit
def kernel(hidden_states, router_w, ws, w2s):
    b, s, h = hidden_states.shape
    x = hidden_states.reshape(-1, h)
    ws4 = ws.reshape(E, 2, I, H)
    out = pl.pallas_call(
        _moe_kernel,
        grid=(E, NI),
        in_specs=[
            pl.BlockSpec((T, H), lambda e, ti: (0, 0)),
            pl.BlockSpec((E, H), lambda e, ti: (0, 0)),
            pl.BlockSpec((1, 1, TI, H), lambda e, ti: (e, 0, ti, 0)),
            pl.BlockSpec((1, 1, TI, H), lambda e, ti: (e, 1, ti, 0)),
            pl.BlockSpec((1, H, TI), lambda e, ti: (e, 0, ti)),
        ],
        out_specs=pl.BlockSpec((T, H), lambda e, ti: (0, 0)),
        out_shape=jax.ShapeDtypeStruct((T, H), jnp.float32),
        scratch_shapes=[pltpu.VMEM((T, E), jnp.float32)],
    )(x, router_w, ws4, ws4, w2s)
    return out.reshape(b, s, h)


# speedup vs baseline: 1.0178x; 1.0178x over previous
import functools

import jax
import jax.numpy as jnp
from jax.experimental import pallas as pl
from jax.experimental.pallas import tpu as pltpu

B, S, H, E, I, TOP_K = 8, 16, 1024, 16, 2048, 2
T = B * S
TI = 1024
NI = I // TI


def _moe_kernel(x_ref, rw_ref, wgu_ref, w2_ref, out_ref, combine_ref):
    e = pl.program_id(0)
    ti = pl.program_id(1)

    @pl.when(jnp.logical_and(e == 0, ti == 0))
    def _routing():
        x = x_ref[...]
        logits = jax.lax.dot_general(
            x, rw_ref[...], (((1,), (1,)), ((), ())),
            preferred_element_type=jnp.float32)
        m = jnp.max(logits, axis=-1, keepdims=True)
        unnorm = jnp.exp(logits - m)
        probs = unnorm / jnp.sum(unnorm, axis=-1, keepdims=True)
        iota = jax.lax.broadcasted_iota(jnp.int32, (T, E), 1)
        m1 = jnp.max(probs, axis=-1, keepdims=True)
        i1 = jnp.min(jnp.where(probs == m1, iota, E), axis=-1, keepdims=True)
        sel1 = iota == i1
        probs2 = jnp.where(sel1, -jnp.inf, probs)
        m2 = jnp.max(probs2, axis=-1, keepdims=True)
        i2 = jnp.min(jnp.where(probs2 == m2, iota, E), axis=-1, keepdims=True)
        sel = sel1 | (iota == i2)
        combine_ref[...] = jnp.where(sel, probs, 0.0)
        out_ref[...] = jnp.zeros_like(out_ref)

    x = x_ref[...].astype(jnp.bfloat16)
    g = jax.lax.dot_general(x, wgu_ref[0, 0].astype(jnp.bfloat16),
                            (((1,), (1,)), ((), ())),
                            preferred_element_type=jnp.float32)
    u = jax.lax.dot_general(x, wgu_ref[0, 1].astype(jnp.bfloat16),
                            (((1,), (1,)), ((), ())),
                            preferred_element_type=jnp.float32)
    a = (g * jax.nn.sigmoid(g)) * u
    iota = jax.lax.broadcasted_iota(jnp.int32, (T, E), 1)
    col = jnp.sum(jnp.where(iota == e, combine_ref[...], 0.0), axis=1,
                  keepdims=True)
    a = (a * col).astype(jnp.bfloat16)
    o = jax.lax.dot_general(a, w2_ref[0].astype(jnp.bfloat16),
                            (((1,), (1,)), ((), ())),
                            preferred_element_type=jnp.float32)
    out_ref[...] += o


@jax.jit
def kernel(hidden_states, router_w, ws, w2s):
    b, s, h = hidden_states.shape
    x = hidden_states.reshape(-1, h)
    ws4 = ws.reshape(E, 2, I, H)
    out = pl.pallas_call(
        _moe_kernel,
        grid=(E, NI),
        in_specs=[
            pl.BlockSpec((T, H), lambda e, ti: (0, 0)),
            pl.BlockSpec((E, H), lambda e, ti: (0, 0)),
            pl.BlockSpec((1, 2, TI, H), lambda e, ti: (e, 0, ti, 0)),
            pl.BlockSpec((1, H, TI), lambda e, ti: (e, 0, ti)),
        ],
        out_specs=pl.BlockSpec((T, H), lambda e, ti: (0, 0)),
        out_shape=jax.ShapeDtypeStruct((T, H), jnp.float32),
        scratch_shapes=[pltpu.VMEM((T, E), jnp.float32)],
    )(x, router_w, ws4, w2s)
    return out.reshape(b, s, h)


# speedup vs baseline: 1.0184x; 1.0006x over previous
import jax
import jax.numpy as jnp
from jax.experimental import pallas as pl
from jax.experimental.pallas import tpu as pltpu

B, S, H, E, I, TOP_K = 8, 16, 1024, 16, 2048, 2
T = B * S
TI = 1024
NI = I // TI


def _moe_kernel(x_ref, rw_ref, wgu_ref, w2_ref, out_ref, combine_ref):
    e = pl.program_id(0)
    ti = pl.program_id(1)

    @pl.when(jnp.logical_and(e == 0, ti == 0))
    def _routing():
        x = x_ref[...]
        logits = jax.lax.dot_general(
            x, rw_ref[...], (((1,), (1,)), ((), ())),
            preferred_element_type=jnp.float32)
        m = jnp.max(logits, axis=-1, keepdims=True)
        unnorm = jnp.exp(logits - m)
        probs = unnorm / jnp.sum(unnorm, axis=-1, keepdims=True)
        iota = jax.lax.broadcasted_iota(jnp.int32, (T, E), 1)
        m1 = jnp.max(probs, axis=-1, keepdims=True)
        i1 = jnp.min(jnp.where(probs == m1, iota, E), axis=-1, keepdims=True)
        sel1 = iota == i1
        probs2 = jnp.where(sel1, -jnp.inf, probs)
        m2 = jnp.max(probs2, axis=-1, keepdims=True)
        i2 = jnp.min(jnp.where(probs2 == m2, iota, E), axis=-1, keepdims=True)
        sel = sel1 | (iota == i2)
        combine_ref[...] = jnp.where(sel, probs, 0.0)
        out_ref[...] = jnp.zeros_like(out_ref)

    x = x_ref[...].astype(jnp.bfloat16)
    g = jax.lax.dot_general(x, wgu_ref[0, 0].astype(jnp.bfloat16),
                            (((1,), (1,)), ((), ())),
                            preferred_element_type=jnp.float32)
    u = jax.lax.dot_general(x, wgu_ref[0, 1].astype(jnp.bfloat16),
                            (((1,), (1,)), ((), ())),
                            preferred_element_type=jnp.float32)
    a = (g * jax.nn.sigmoid(g)) * u
    iota = jax.lax.broadcasted_iota(jnp.int32, (T, E), 1)
    col = jnp.sum(jnp.where(iota == e, combine_ref[...], 0.0), axis=1,
                  keepdims=True)
    a = (a * col).astype(jnp.bfloat16)
    o = jax.lax.dot_general(a, w2_ref[0].astype(jnp.bfloat16),
                            (((1,), (1,)), ((), ())),
                            preferred_element_type=jnp.float32)
    out_ref[...] += o


@jax.jit
def kernel(hidden_states, router_w, ws, w2s):
    b, s, h = hidden_states.shape
    x = hidden_states.reshape(-1, h)
    ws4 = ws.reshape(E, 2, I, H)
    out = pl.pallas_call(
        _moe_kernel,
        grid=(E, NI),
        in_specs=[
            pl.BlockSpec((T, H), lambda e, ti: (0, 0)),
            pl.BlockSpec((E, H), lambda e, ti: (0, 0)),
            pl.BlockSpec((1, 2, TI, H), lambda e, ti: (e, 0, ti, 0)),
            pl.BlockSpec((1, H, TI), lambda e, ti: (e, 0, ti)),
        ],
        out_specs=pl.BlockSpec((T, H), lambda e, ti: (0, 0)),
        out_shape=jax.ShapeDtypeStruct((T, H), jnp.float32),
        scratch_shapes=[pltpu.VMEM((T, E), jnp.float32)],
    )(x, router_w, ws4, w2s)
    return out.reshape(b, s, h)
